# Initial kernel scaffold; baseline (speedup 1.0000x reference)
import functools

import jax
import jax.numpy as jnp
from jax import lax
from jax.experimental import pallas as pl
from jax.experimental.pallas import tpu as pltpu
from jax.experimental.pallas import tpu_sc as plsc

DIM = 64
SUBMAX = 50
NSAMP = 20
LANES = 16
NCHUNKS_D = DIM // LANES


@functools.lru_cache(maxsize=None)
def _build_sisg(B: int, n_workers: int, C: int):
    b_per_w = B // n_workers
    n_chunks = b_per_w // C
    mesh = plsc.VectorSubcoreMesh(core_axis_name="c", subcore_axis_name="s")

    @functools.partial(
        pl.kernel,
        out_type=jax.ShapeDtypeStruct((B * NSAMP,), jnp.float32),
        mesh=mesh,
        scratch_types=[
            pltpu.VMEM((b_per_w,), jnp.int32),
            pltpu.VMEM((b_per_w * SUBMAX,), jnp.int32),
            pltpu.VMEM((b_per_w * NSAMP,), jnp.int32),
            pltpu.VMEM((b_per_w,), jnp.float32),
            pltpu.VMEM((C, DIM), jnp.float32),
            pltpu.VMEM((C * SUBMAX, DIM), jnp.float32),
            pltpu.VMEM((C * NSAMP, DIM), jnp.float32),
            pltpu.VMEM((b_per_w * NSAMP,), jnp.float32),
            pltpu.SemaphoreType.DMA,
        ],
    )
    def sisg(tgt_hbm, sub_hbm, len_hbm, samp_hbm, in_emb, out_emb, out_hbm,
             tgt_i, sub_i, samp_i, inv_v, tgt_r, sub_r, samp_r, sc_v, sem):
        nc = 2
        wid = lax.axis_index("s") * nc + lax.axis_index("c")
        base = wid * b_per_w

        pltpu.sync_copy(tgt_hbm.at[pl.ds(base, b_per_w)], tgt_i)
        pltpu.sync_copy(sub_hbm.at[pl.ds(base * SUBMAX, b_per_w * SUBMAX)], sub_i)
        pltpu.sync_copy(samp_hbm.at[pl.ds(base * NSAMP, b_per_w * NSAMP)], samp_i)
        pltpu.sync_copy(len_hbm.at[pl.ds(base, b_per_w)], inv_v)

        def inv_body(i, _):
            v = inv_v[pl.ds(i * LANES, LANES)]
            inv_v[pl.ds(i * LANES, LANES)] = 1.0 / v
            return 0
        lax.fori_loop(0, b_per_w // LANES, inv_body, 0)

        def chunk_body(c, _):
            r0 = pl.multiple_of(c * C, C)
            cp1 = pltpu.async_copy(in_emb.at[tgt_i.at[pl.ds(r0, C)]], tgt_r, sem)
            cp2 = pltpu.async_copy(
                in_emb.at[sub_i.at[pl.ds(r0 * SUBMAX, C * SUBMAX)]], sub_r, sem)
            cp3 = pltpu.async_copy(
                out_emb.at[samp_i.at[pl.ds(r0 * NSAMP, C * NSAMP)]], samp_r, sem)
            cp1.wait()
            cp2.wait()
            cp3.wait()

            def row_body(r, _):
                inv = inv_v[r0 + r]

                def sub_body(j, acc):
                    row = r * SUBMAX + j
                    return tuple(acc[k] + sub_r[row, pl.ds(LANES * k, LANES)]
                                 for k in range(NCHUNKS_D))
                acc = lax.fori_loop(
                    0, SUBMAX, sub_body,
                    tuple(jnp.zeros((LANES,), jnp.float32)
                          for _ in range(NCHUNKS_D)))
                w = tuple(tgt_r[r, pl.ds(LANES * k, LANES)] + acc[k] * inv
                          for k in range(NCHUNKS_D))

                def samp_body(s, _):
                    row = r * NSAMP + s
                    p = w[0] * samp_r[row, pl.ds(0, LANES)]
                    for k in range(1, NCHUNKS_D):
                        p = p + w[k] * samp_r[row, pl.ds(LANES * k, LANES)]
                    sc_v[(r0 + r) * NSAMP + s] = jnp.sum(p)
                    return 0
                lax.fori_loop(0, NSAMP, samp_body, 0)
                return 0
            lax.fori_loop(0, C, row_body, 0)
            return 0
        lax.fori_loop(0, n_chunks, chunk_body, 0)

        def sig_body(i, _):
            v = sc_v[pl.ds(i * LANES, LANES)]
            sc_v[pl.ds(i * LANES, LANES)] = 1.0 / (1.0 + jnp.exp(-v))
            return 0
        lax.fori_loop(0, b_per_w * NSAMP // LANES, sig_body, 0)

        pltpu.sync_copy(sc_v, out_hbm.at[pl.ds(base * NSAMP, b_per_w * NSAMP)])

    return sisg


def kernel(targets, subwords, subword_length, samples, word_in_emb, word_out_emb):
    B = targets.shape[0]
    tgt = targets.astype(jnp.int32)
    sub = subwords.astype(jnp.int32).reshape(-1)
    samp = samples.astype(jnp.int32).reshape(-1)
    out = _build_sisg(B, 32, 16)(
        tgt, sub, subword_length.astype(jnp.float32), samp,
        word_in_emb, word_out_emb)
    return out.reshape(B, NSAMP)


# speedup vs baseline: 1.3489x; 1.3489x over previous
import functools

import jax
import jax.numpy as jnp
from jax import lax
from jax.experimental import pallas as pl
from jax.experimental.pallas import tpu as pltpu
from jax.experimental.pallas import tpu_sc as plsc

DIM = 64
SUBMAX = 50
NSAMP = 20
LANES = 16
NCHUNKS_D = DIM // LANES


@functools.lru_cache(maxsize=None)
def _build_sisg(B: int, n_workers: int, C: int):
    b_per_w = B // n_workers
    n_chunks = b_per_w // C
    mesh = plsc.VectorSubcoreMesh(core_axis_name="c", subcore_axis_name="s")

    @functools.partial(
        pl.kernel,
        out_type=jax.ShapeDtypeStruct((B * NSAMP,), jnp.float32),
        mesh=mesh,
        scratch_types=[
            pltpu.VMEM((b_per_w,), jnp.int32),
            pltpu.VMEM((b_per_w * SUBMAX,), jnp.int32),
            pltpu.VMEM((b_per_w * NSAMP,), jnp.int32),
            pltpu.VMEM((b_per_w,), jnp.float32),
            pltpu.VMEM((C, DIM), jnp.float32),
            pltpu.VMEM((C * SUBMAX, DIM), jnp.float32),
            pltpu.VMEM((C * NSAMP, DIM), jnp.float32),
            pltpu.VMEM((b_per_w * NSAMP,), jnp.float32),
            pltpu.SemaphoreType.DMA,
        ],
        compiler_params=pltpu.CompilerParams(
            needs_layout_passes=False, use_tc_tiling_on_sc=False),
    )
    def sisg(tgt_hbm, sub_hbm, len_hbm, samp_hbm, in_emb, out_emb, out_hbm,
             tgt_i, sub_i, samp_i, inv_v, tgt_r, sub_r, samp_r, sc_v, sem):
        nc = 2
        wid = lax.axis_index("s") * nc + lax.axis_index("c")
        base = wid * b_per_w

        pltpu.sync_copy(tgt_hbm.at[pl.ds(base, b_per_w)], tgt_i)
        pltpu.sync_copy(sub_hbm.at[pl.ds(base * SUBMAX, b_per_w * SUBMAX)], sub_i)
        pltpu.sync_copy(samp_hbm.at[pl.ds(base * NSAMP, b_per_w * NSAMP)], samp_i)
        pltpu.sync_copy(len_hbm.at[pl.ds(base, b_per_w)], inv_v)

        def inv_body(i, _):
            v = inv_v[pl.ds(i * LANES, LANES)]
            inv_v[pl.ds(i * LANES, LANES)] = 1.0 / v
            return 0
        lax.fori_loop(0, b_per_w // LANES, inv_body, 0)

        def chunk_body(c, _):
            r0 = pl.multiple_of(c * C, C)
            cp1 = pltpu.async_copy(in_emb.at[tgt_i.at[pl.ds(r0, C)]], tgt_r, sem)
            cp2 = pltpu.async_copy(
                in_emb.at[sub_i.at[pl.ds(r0 * SUBMAX, C * SUBMAX)]], sub_r, sem)
            cp3 = pltpu.async_copy(
                out_emb.at[samp_i.at[pl.ds(r0 * NSAMP, C * NSAMP)]], samp_r, sem)
            cp1.wait()
            cp2.wait()
            cp3.wait()

            lane = jnp.arange(LANES, dtype=jnp.int32)
            lane0 = lane == 0

            def row_body(r, _):
                inv = plsc.load_gather(
                    inv_v, [jnp.full((LANES,), r0 + r, jnp.int32)])

                def sub_body(j, acc):
                    row = r * SUBMAX + j
                    return tuple(acc[k] + sub_r[row, pl.ds(LANES * k, LANES)]
                                 for k in range(NCHUNKS_D))
                acc = lax.fori_loop(
                    0, SUBMAX, sub_body,
                    tuple(jnp.zeros((LANES,), jnp.float32)
                          for _ in range(NCHUNKS_D)))
                w = tuple(tgt_r[r, pl.ds(LANES * k, LANES)] + acc[k] * inv
                          for k in range(NCHUNKS_D))

                def samp_body(s, _):
                    row = r * NSAMP + s
                    p = w[0] * samp_r[row, pl.ds(0, LANES)]
                    for k in range(1, NCHUNKS_D):
                        p = p + w[k] * samp_r[row, pl.ds(LANES * k, LANES)]
                    ssum = jnp.full((LANES,), jnp.sum(p))
                    plsc.store_scatter(
                        sc_v,
                        [jnp.full((LANES,), (r0 + r) * NSAMP + s, jnp.int32)],
                        ssum, mask=lane0)
                    return 0
                lax.fori_loop(0, NSAMP, samp_body, 0)
                return 0
            lax.fori_loop(0, C, row_body, 0)
            return 0
        lax.fori_loop(0, n_chunks, chunk_body, 0)

        def sig_body(i, _):
            v = sc_v[pl.ds(i * LANES, LANES)]
            sc_v[pl.ds(i * LANES, LANES)] = 1.0 / (1.0 + jnp.exp(-v))
            return 0
        lax.fori_loop(0, b_per_w * NSAMP // LANES, sig_body, 0)

        pltpu.sync_copy(sc_v, out_hbm.at[pl.ds(base * NSAMP, b_per_w * NSAMP)])

    return sisg


def kernel(targets, subwords, subword_length, samples, word_in_emb, word_out_emb):
    B = targets.shape[0]
    tgt = targets.astype(jnp.int32)
    sub = subwords.astype(jnp.int32).reshape(-1)
    samp = samples.astype(jnp.int32).reshape(-1)
    out = _build_sisg(B, 32, 16)(
        tgt, sub, subword_length.astype(jnp.float32), samp,
        word_in_emb, word_out_emb)
    return out.reshape(B, NSAMP)


# speedup vs baseline: 1.4270x; 1.0579x over previous
import functools

import jax
import jax.numpy as jnp
from jax import lax
from jax.experimental import pallas as pl
from jax.experimental.pallas import tpu as pltpu
from jax.experimental.pallas import tpu_sc as plsc

DIM = 64
SUBMAX = 50
NSAMP = 20
LANES = 16
NCHUNKS_D = DIM // LANES


@functools.lru_cache(maxsize=None)
def _build_sisg(B: int, n_workers: int, C: int):
    b_per_w = B // n_workers
    n_chunks = b_per_w // C
    assert n_chunks % 2 == 0
    mesh = plsc.VectorSubcoreMesh(core_axis_name="c", subcore_axis_name="s")

    buf = lambda: (pltpu.VMEM((C, DIM), jnp.float32),
                   pltpu.VMEM((C * SUBMAX, DIM), jnp.float32),
                   pltpu.VMEM((C * NSAMP, DIM), jnp.float32),
                   pltpu.SemaphoreType.DMA)

    @functools.partial(
        pl.kernel,
        out_type=jax.ShapeDtypeStruct((B * NSAMP,), jnp.float32),
        mesh=mesh,
        scratch_types=[
            pltpu.VMEM((b_per_w,), jnp.int32),
            pltpu.VMEM((b_per_w * SUBMAX,), jnp.int32),
            pltpu.VMEM((b_per_w * NSAMP,), jnp.int32),
            pltpu.VMEM((b_per_w,), jnp.float32),
            *buf(), *buf(),
            pltpu.VMEM((C * NSAMP * LANES,), jnp.float32),
            pltpu.VMEM((b_per_w * NSAMP,), jnp.float32),
        ],
        compiler_params=pltpu.CompilerParams(
            needs_layout_passes=False, use_tc_tiling_on_sc=False),
    )
    def sisg(tgt_hbm, sub_hbm, len_hbm, samp_hbm, in_emb, out_emb, out_hbm,
             tgt_i, sub_i, samp_i, inv_v,
             tgt_r0, sub_r0, samp_r0, sem0,
             tgt_r1, sub_r1, samp_r1, sem1,
             part_v, sc_v):
        nc = 2
        wid = lax.axis_index("s") * nc + lax.axis_index("c")
        base = wid * b_per_w
        bufs = ((tgt_r0, sub_r0, samp_r0, sem0),
                (tgt_r1, sub_r1, samp_r1, sem1))
        lane = jnp.arange(LANES, dtype=jnp.int32)

        pltpu.sync_copy(tgt_hbm.at[pl.ds(base, b_per_w)], tgt_i)
        pltpu.sync_copy(sub_hbm.at[pl.ds(base * SUBMAX, b_per_w * SUBMAX)], sub_i)
        pltpu.sync_copy(samp_hbm.at[pl.ds(base * NSAMP, b_per_w * NSAMP)], samp_i)
        pltpu.sync_copy(len_hbm.at[pl.ds(base, b_per_w)], inv_v)

        def inv_body(i, _):
            v = inv_v[pl.ds(i * LANES, LANES)]
            inv_v[pl.ds(i * LANES, LANES)] = 1.0 / v
            return 0
        lax.fori_loop(0, b_per_w // LANES, inv_body, 0)

        def issue(c, bi):
            tr, sr, pr, sem = bufs[bi]
            r0 = pl.multiple_of(c * C, C)
            pltpu.async_copy(in_emb.at[tgt_i.at[pl.ds(r0, C)]], tr, sem)
            pltpu.async_copy(
                in_emb.at[sub_i.at[pl.ds(r0 * SUBMAX, C * SUBMAX)]], sr, sem)
            pltpu.async_copy(
                out_emb.at[samp_i.at[pl.ds(r0 * NSAMP, C * NSAMP)]], pr, sem)

        def drain(c, bi):
            tr, sr, pr, sem = bufs[bi]
            r0 = pl.multiple_of(c * C, C)
            pltpu.make_async_copy(
                in_emb.at[tgt_i.at[pl.ds(r0, C)]], tr, sem).wait()
            pltpu.make_async_copy(
                in_emb.at[sub_i.at[pl.ds(r0 * SUBMAX, C * SUBMAX)]],
                sr, sem).wait()
            pltpu.make_async_copy(
                out_emb.at[samp_i.at[pl.ds(r0 * NSAMP, C * NSAMP)]],
                pr, sem).wait()

        def compute(c, bi):
            tr, sr, pr, _ = bufs[bi]
            r0 = pl.multiple_of(c * C, C)

            def row_body(r, _):
                inv = plsc.load_gather(
                    inv_v, [jnp.full((LANES,), r0 + r, jnp.int32)])

                def sub_body(j, acc):
                    row = r * SUBMAX + j
                    return tuple(acc[k] + sr[row, pl.ds(LANES * k, LANES)]
                                 for k in range(NCHUNKS_D))
                acc = lax.fori_loop(
                    0, SUBMAX, sub_body,
                    tuple(jnp.zeros((LANES,), jnp.float32)
                          for _ in range(NCHUNKS_D)),
                    unroll=10)
                w = tuple(tr[r, pl.ds(LANES * k, LANES)] + acc[k] * inv
                          for k in range(NCHUNKS_D))

                for s in range(NSAMP):
                    row = r * NSAMP + s
                    p = w[0] * pr[row, pl.ds(0, LANES)]
                    for k in range(1, NCHUNKS_D):
                        p = p + w[k] * pr[row, pl.ds(LANES * k, LANES)]
                    part_v[pl.ds(row * LANES, LANES)] = p
                return 0
            lax.fori_loop(0, C, row_body, 0)

            for g in range(C * NSAMP // LANES):
                bidx = lane * LANES + (g * LANES * LANES)
                sv = plsc.load_gather(part_v, [bidx])
                for cc in range(1, LANES):
                    sv = sv + plsc.load_gather(part_v, [bidx + cc])
                sv = 1.0 / (1.0 + jnp.exp(-sv))
                sc_v[pl.ds(c * C * NSAMP + g * LANES, LANES)] = sv

        issue(0, 0)

        def big_body(c2, _):
            c = c2 * 2
            drain(c, 0)
            issue(c + 1, 1)
            compute(c, 0)
            drain(c + 1, 1)

            @pl.when(c2 + 1 < n_chunks // 2)
            def _():
                issue(c + 2, 0)
            compute(c + 1, 1)
            return 0
        lax.fori_loop(0, n_chunks // 2, big_body, 0)

        pltpu.sync_copy(sc_v, out_hbm.at[pl.ds(base * NSAMP, b_per_w * NSAMP)])

    return sisg


def kernel(targets, subwords, subword_length, samples, word_in_emb, word_out_emb):
    B = targets.shape[0]
    tgt = targets.astype(jnp.int32)
    sub = subwords.astype(jnp.int32).reshape(-1)
    samp = samples.astype(jnp.int32).reshape(-1)
    out = _build_sisg(B, 32, 8)(
        tgt, sub, subword_length.astype(jnp.float32), samp,
        word_in_emb, word_out_emb)
    return out.reshape(B, NSAMP)
